# Initial kernel scaffold; baseline (speedup 1.0000x reference)
import jax
import jax.numpy as jnp
from jax.experimental import pallas as pl


def kernel(sal_map, fix):
    raise NotImplementedError("write your pallas kernel here")


# speedup vs baseline: 1.3035x; 1.3035x over previous
import jax
import jax.numpy as jnp
from jax.experimental import pallas as pl
from jax.experimental.pallas import tpu as pltpu


def _tc_body(sal_ref, fix_ref, out_ref):
    i = pl.program_id(0)
    s = sal_ref[...]
    f = fix_ref[...]
    m = f > 0.1
    ssum = jnp.sum(s)
    ssq = jnp.sum(s * s)
    msum = jnp.sum(jnp.where(m, s, 0.0))
    cnt = jnp.sum(jnp.where(m, 1.0, 0.0))

    @pl.when(i == 0)
    def _init():
        out_ref[0] = 0.0
        out_ref[1] = 0.0
        out_ref[2] = 0.0
        out_ref[3] = 0.0

    out_ref[0] += ssum
    out_ref[1] += ssq
    out_ref[2] += msum
    out_ref[3] += cnt


def kernel(sal_map, fix):
    b, h, w = sal_map.shape
    n = b * h * w
    partials = pl.pallas_call(
        _tc_body,
        grid=(b,),
        in_specs=[
            pl.BlockSpec((1, h, w), lambda i: (i, 0, 0)),
            pl.BlockSpec((1, h, w), lambda i: (i, 0, 0)),
        ],
        out_specs=pl.BlockSpec(memory_space=pltpu.SMEM),
        out_shape=jax.ShapeDtypeStruct((4,), jnp.float32),
    )(sal_map, fix)
    ssum, ssq, msum, cnt = partials[0], partials[1], partials[2], partials[3]
    nf = jnp.float32(n)
    mean = ssum / nf
    var = (ssq - nf * mean * mean) / (nf - 1.0)
    std = jnp.sqrt(var)
    return (msum - cnt * mean) / (std * cnt)
